# baseline (speedup 1.0000x reference)
import functools

import jax
import jax.numpy as jnp
from jax import lax
from jax.experimental import pallas as pl
from jax.experimental.pallas import tpu as pltpu
from jax.experimental.pallas import tpu_sc as plsc

NUM_FIELDS = 26
VOCAB = 100000
EMB_DIM = 16
BATCH = 4096

_NC = 2
_NS = 16
_LANES = 16
_NW = _NC * _NS
_TOTAL = BATCH * NUM_FIELDS
_PER_W = _TOTAL // _NW
_IDX_ROWS = _PER_W // 128
_STEPS = _PER_W // _LANES


def _sc_gather(tab_hbm, idx_hbm, out_hbm, idx_v, rows_v, sem):
    wid = lax.axis_index("s") * _NC + lax.axis_index("c")
    base = wid * _PER_W

    pltpu.sync_copy(idx_hbm.at[wid], idx_v)

    f0 = lax.iota(jnp.int32, _LANES)

    def offset_body(i, f):
        j = i // 8
        c = i - j * 8
        sl = idx_v[j, pl.ds(c * _LANES, _LANES)]
        idx_v[j, pl.ds(c * _LANES, _LANES)] = sl + f * VOCAB
        t = f + _LANES
        return jnp.where(t >= NUM_FIELDS, t - NUM_FIELDS, t)

    lax.fori_loop(0, _STEPS, offset_body, f0)

    def dma_body(j, carry):
        pltpu.async_copy(
            tab_hbm.at[idx_v.at[j]],
            rows_v.at[pl.ds(j * 128, 128)],
            sem,
        ).wait()
        return carry

    lax.fori_loop(0, _IDX_ROWS, dma_body, 0)

    pltpu.sync_copy(rows_v, out_hbm.at[pl.ds(base, _PER_W)])


@jax.jit
def _impl(indices, tables):
    tab = tables.reshape(NUM_FIELDS * VOCAB, EMB_DIM)
    idx = indices.reshape(_NW, _IDX_ROWS, 128)
    mesh = plsc.VectorSubcoreMesh(core_axis_name="c", subcore_axis_name="s")
    run = pl.kernel(
        _sc_gather,
        out_type=jax.ShapeDtypeStruct((_TOTAL, EMB_DIM), jnp.float32),
        mesh=mesh,
        compiler_params=pltpu.CompilerParams(use_tc_tiling_on_sc=False),
        scratch_types=[
            pltpu.VMEM((_IDX_ROWS, 128), jnp.int32),
            pltpu.VMEM((_PER_W, EMB_DIM), jnp.float32),
            pltpu.SemaphoreType.DMA,
        ],
    )
    out = run(tab, idx)
    return out.reshape(BATCH, NUM_FIELDS * EMB_DIM)


def kernel(indices, tables):
    return _impl(indices, tables)
